# baseline (speedup 1.0000x reference)
import functools

import jax
import jax.numpy as jnp
from jax import lax
from jax.experimental import pallas as pl
from jax.experimental.pallas import tpu as pltpu
from jax.experimental.pallas import tpu_sc as plsc

_N = 10000
_E = 320000
_G = 256
_H = 128

_NC = 2
_NS = 16
_NW = _NC * _NS
_EPW = _E // _NW
_W = 80

_BE = 2560
_CHUNKS = (64000,) * 5
_BN = 1000


@functools.lru_cache(maxsize=None)
def _mesh():
    return plsc.VectorSubcoreMesh(core_axis_name="c", subcore_axis_name="s")


def _pre_body(nf_ref, lat9_ref, w1a_ref, w1b_ref, w1c_ref, b1_ref,
              p_ref, q_ref, rb_ref):
    x = nf_ref[...]
    p_ref[...] = jnp.dot(x, w1a_ref[...],
                         preferred_element_type=jnp.float32).astype(jnp.bfloat16)
    q_ref[...] = jnp.dot(x, w1b_ref[...],
                         preferred_element_type=jnp.float32).astype(jnp.bfloat16)

    @pl.when(pl.program_id(0) == 0)
    def _():
        lat9 = lat9_ref[...]
        cols = [lat9[:, k:k + 1] for k in range(9)]
        acc = jnp.broadcast_to(b1_ref[...], (_G, _H))
        for i in range(3):
            for j in range(3):
                ip = (cols[3 * i + 0] * cols[3 * j + 0]
                      + cols[3 * i + 1] * cols[3 * j + 1]
                      + cols[3 * i + 2] * cols[3 * j + 2])
                acc = acc + ip * w1c_ref[3 * i + j:3 * i + j + 1, :]
        rb_ref[...] = acc.astype(jnp.bfloat16)


def _pre(nf, lat9, w1a, w1b, w1c, b1r):
    grid = (_N // _BN,)
    return pl.pallas_call(
        _pre_body,
        grid=grid,
        in_specs=[
            pl.BlockSpec((_BN, _H), lambda i: (i, 0)),
            pl.BlockSpec((_G, 9), lambda i: (0, 0)),
            pl.BlockSpec((_H, _H), lambda i: (0, 0)),
            pl.BlockSpec((_H, _H), lambda i: (0, 0)),
            pl.BlockSpec((9, _H), lambda i: (0, 0)),
            pl.BlockSpec((1, _H), lambda i: (0, 0)),
        ],
        out_specs=[
            pl.BlockSpec((_BN, _H), lambda i: (i, 0)),
            pl.BlockSpec((_BN, _H), lambda i: (i, 0)),
            pl.BlockSpec((_G, _H), lambda i: (0, 0)),
        ],
        out_shape=[
            jax.ShapeDtypeStruct((_N, _H), jnp.bfloat16),
            jax.ShapeDtypeStruct((_N, _H), jnp.bfloat16),
            jax.ShapeDtypeStruct((_G, _H), jnp.bfloat16),
        ],
    )(nf, lat9, w1a, w1b, w1c, b1r)


def _gather_body(epw, nwin,
                 p_hbm, q_hbm, rb_hbm, fc_hbm, src_hbm, dst_hbm, e2g_hbm,
                 ab_hbm, cx_hbm,
                 si_v, di_v, gi_v,
                 a0_v, b0_v, c0_v, xi0_v, xj0_v,
                 a1_v, b1_v, c1_v, xi1_v, xj1_v,
                 sem0, sem1):
    wid = lax.axis_index("s") * _NC + lax.axis_index("c")
    base = wid * epw
    ab0_v = (a0_v, b0_v, c0_v, xi0_v, xj0_v)
    ab1_v = (a1_v, b1_v, c1_v, xi1_v, xj1_v)

    pltpu.sync_copy(src_hbm.at[pl.ds(base, epw)], si_v)
    pltpu.sync_copy(dst_hbm.at[pl.ds(base, epw)], di_v)
    pltpu.sync_copy(e2g_hbm.at[pl.ds(base, epw)], gi_v)

    def issue(w, bufs, sem):
        a_v, b_v, c_v, xi_v, xj_v = bufs
        sl = pl.ds(w * _W, _W)
        pltpu.async_copy(p_hbm.at[si_v.at[sl]], a_v, sem)
        pltpu.async_copy(q_hbm.at[di_v.at[sl]], b_v, sem)
        pltpu.async_copy(rb_hbm.at[gi_v.at[sl]], c_v, sem)
        pltpu.async_copy(fc_hbm.at[si_v.at[sl]], xi_v, sem)
        pltpu.async_copy(fc_hbm.at[di_v.at[sl]], xj_v, sem)

    def drain_store(w, bufs, sem):
        a_v, b_v, c_v, xi_v, xj_v = bufs
        sl = pl.ds(w * _W, _W)
        pltpu.make_async_copy(p_hbm.at[si_v.at[sl]], a_v, sem).wait()
        pltpu.make_async_copy(q_hbm.at[di_v.at[sl]], b_v, sem).wait()
        pltpu.make_async_copy(rb_hbm.at[gi_v.at[sl]], c_v, sem).wait()
        pltpu.make_async_copy(fc_hbm.at[si_v.at[sl]], xi_v, sem).wait()
        pltpu.make_async_copy(fc_hbm.at[di_v.at[sl]], xj_v, sem).wait()
        rows = pl.ds(base + w * _W, _W)
        pltpu.sync_copy(a_v, ab_hbm.at[rows, pl.ds(0, 64)])
        pltpu.sync_copy(b_v, ab_hbm.at[rows, pl.ds(64, 64)])
        pltpu.sync_copy(c_v, cx_hbm.at[rows, pl.ds(0, 64)])
        pltpu.sync_copy(xi_v, cx_hbm.at[rows, pl.ds(64, 16)])
        pltpu.sync_copy(xj_v, cx_hbm.at[rows, pl.ds(80, 16)])

    issue(0, ab0_v, sem0)

    if nwin % 2 == 1:
        @pl.loop(0, nwin - 1, step=2)
        def _(w):
            issue(w + 1, ab1_v, sem1)
            drain_store(w, ab0_v, sem0)
            issue(w + 2, ab0_v, sem0)
            drain_store(w + 1, ab1_v, sem1)

        drain_store(nwin - 1, ab0_v, sem0)
    else:
        @pl.loop(0, nwin - 2, step=2)
        def _(w):
            issue(w + 1, ab1_v, sem1)
            drain_store(w, ab0_v, sem0)
            issue(w + 2, ab0_v, sem0)
            drain_store(w + 1, ab1_v, sem1)

        issue(nwin - 1, ab1_v, sem1)
        drain_store(nwin - 2, ab0_v, sem0)
        drain_store(nwin - 1, ab1_v, sem1)


@functools.lru_cache(maxsize=None)
def _gather_kernel(n_edges):
    epw = n_edges // _NW
    nwin = epw // _W
    return pl.kernel(
        functools.partial(_gather_body, epw, nwin),
        mesh=_mesh(),
        out_type=(
            jax.ShapeDtypeStruct((n_edges, _H), jnp.int32),
            jax.ShapeDtypeStruct((n_edges, _H), jnp.int32),
        ),
        scratch_types=[
            pltpu.VMEM((epw,), jnp.int32),
            pltpu.VMEM((epw,), jnp.int32),
            pltpu.VMEM((epw,), jnp.int32),
            pltpu.VMEM((_W, 64), jnp.int32),
            pltpu.VMEM((_W, 64), jnp.int32),
            pltpu.VMEM((_W, 64), jnp.int32),
            pltpu.VMEM((_W, 16), jnp.int32),
            pltpu.VMEM((_W, 16), jnp.int32),
            pltpu.VMEM((_W, 64), jnp.int32),
            pltpu.VMEM((_W, 64), jnp.int32),
            pltpu.VMEM((_W, 64), jnp.int32),
            pltpu.VMEM((_W, 16), jnp.int32),
            pltpu.VMEM((_W, 16), jnp.int32),
            pltpu.SemaphoreType.DMA,
            pltpu.SemaphoreType.DMA,
        ],
        compiler_params=pltpu.CompilerParams(use_tc_tiling_on_sc=False),
    )


def _aux_body(src_hbm, ones_hbm, zc_hbm, cnt_hbm,
              si_v, ones_v, cacc_sh, sem):
    cid = lax.axis_index("c")
    sid = lax.axis_index("s")
    wid = sid * _NC + cid
    base = wid * _EPW

    @pl.when(sid == 0)
    def _():
        pltpu.sync_copy(zc_hbm, cacc_sh)
    pltpu.sync_copy(ones_hbm, ones_v)
    plsc.subcore_barrier()

    @pl.loop(0, _EPW, step=_W)
    def _(off):
        st = base + off
        pltpu.sync_copy(src_hbm.at[pl.ds(st, _W)], si_v)
        pltpu.sync_copy(ones_v, cacc_sh.at[si_v], add=True)

    plsc.subcore_barrier()

    @pl.when(sid == 0)
    def _():
        pltpu.sync_copy(cacc_sh, cnt_hbm.at[cid])


@functools.lru_cache(maxsize=None)
def _aux_kernel():
    return pl.kernel(
        _aux_body,
        mesh=_mesh(),
        out_type=jax.ShapeDtypeStruct((_NC, _N, 16), jnp.float32),
        scratch_types=[
            pltpu.VMEM((_W,), jnp.int32),
            pltpu.VMEM((_W, 16), jnp.float32),
            pltpu.VMEM_SHARED((_N, 16), jnp.float32),
            pltpu.SemaphoreType.DMA,
        ],
        compiler_params=pltpu.CompilerParams(use_tc_tiling_on_sc=False),
    )


def _edge_body(ab_ref, cx_ref, w1dp_ref, w2a_ref, w2b_ref, b2_ref, out_ref):
    ab = ab_ref[...]
    cx = cx_ref[...]
    mask = jnp.int32(-65536)
    lo_ab = lax.bitcast_convert_type(ab << 16, jnp.float32)
    hi_ab = lax.bitcast_convert_type(ab & mask, jnp.float32)
    lo_c = lax.bitcast_convert_type(cx << 16, jnp.float32)
    hi_c = lax.bitcast_convert_type(cx & mask, jnp.float32)
    xi = lax.bitcast_convert_type(cx[:, 64:80], jnp.float32)
    xj = lax.bitcast_convert_type(cx[:, 80:96], jnp.float32)
    d = xj - xi
    fd = jnp.where(d < 0.0, d + 1.0, d)
    fterm = jnp.dot(fd, w1dp_ref[...], preferred_element_type=jnp.float32)
    zl = (lo_ab[:, :64] + lo_ab[:, 64:] + lo_c[:, :64] + fterm[:, :64])
    zh = (hi_ab[:, :64] + hi_ab[:, 64:] + hi_c[:, :64] + fterm[:, 64:])
    ul = zl * jax.nn.sigmoid(zl)
    uh = zh * jax.nn.sigmoid(zh)
    h2 = (jnp.dot(ul.astype(jnp.bfloat16), w2a_ref[...],
                  preferred_element_type=jnp.float32)
          + jnp.dot(uh.astype(jnp.bfloat16), w2b_ref[...],
                    preferred_element_type=jnp.float32)
          + b2_ref[...])
    out_ref[...] = h2 * jax.nn.sigmoid(h2)


def _edge(ab, cx, w1dp, w2a, w2b, b2r):
    n_edges = ab.shape[0]
    grid = (n_edges // _BE,)
    return pl.pallas_call(
        _edge_body,
        grid=grid,
        in_specs=[
            pl.BlockSpec((_BE, _H), lambda i: (i, 0)),
            pl.BlockSpec((_BE, _H), lambda i: (i, 0)),
            pl.BlockSpec((16, _H), lambda i: (0, 0)),
            pl.BlockSpec((_H // 2, _H), lambda i: (0, 0)),
            pl.BlockSpec((_H // 2, _H), lambda i: (0, 0)),
            pl.BlockSpec((1, _H), lambda i: (0, 0)),
        ],
        out_specs=pl.BlockSpec((_BE, _H), lambda i: (i, 0)),
        out_shape=jax.ShapeDtypeStruct((n_edges, _H), jnp.float32),
    )(ab, cx, w1dp, w2a, w2b, b2r)


def _scatter_body(epw, ef_hbm, src_hbm, z_hbm, part_hbm, idx_v, rows_v,
                  acc_sh, sem):
    cid = lax.axis_index("c")
    sid = lax.axis_index("s")
    wid = sid * _NC + cid
    base = wid * epw

    @pl.when(sid == 0)
    def _():
        pltpu.sync_copy(z_hbm.at[cid], acc_sh)
    plsc.subcore_barrier()

    @pl.loop(0, epw, step=_W)
    def _(off):
        st = base + off
        pltpu.sync_copy(src_hbm.at[pl.ds(st, _W)], idx_v)
        pltpu.sync_copy(ef_hbm.at[pl.ds(st, _W)], rows_v)
        pltpu.sync_copy(rows_v, acc_sh.at[idx_v], add=True)

    plsc.subcore_barrier()

    @pl.when(sid == 0)
    def _():
        pltpu.sync_copy(acc_sh, part_hbm.at[cid])


@functools.lru_cache(maxsize=None)
def _scatter_kernel(n_edges):
    return pl.kernel(
        functools.partial(_scatter_body, n_edges // _NW),
        mesh=_mesh(),
        out_type=jax.ShapeDtypeStruct((_NC, _N, _H), jnp.float32),
        scratch_types=[
            pltpu.VMEM((_W,), jnp.int32),
            pltpu.VMEM((_W, _H), jnp.float32),
            pltpu.VMEM_SHARED((_N, _H), jnp.float32),
            pltpu.SemaphoreType.DMA,
        ],
    )


def _node_body(part_ref, cnt_ref, nf_ref, w3a_ref, w3b_ref, b3_ref, w4_ref,
               b4_ref, out_ref):
    s = part_ref[0] + part_ref[1]
    cnt = cnt_ref[0][:, 0:1] + cnt_ref[1][:, 0:1]
    agg = s / jnp.maximum(cnt, 1.0)
    x = nf_ref[...]
    h = (jnp.dot(x, w3a_ref[...], preferred_element_type=jnp.float32)
         + jnp.dot(agg, w3b_ref[...], preferred_element_type=jnp.float32)
         + b3_ref[...])
    u = h * jax.nn.sigmoid(h)
    h2 = jnp.dot(u, w4_ref[...], preferred_element_type=jnp.float32) + b4_ref[...]
    out_ref[...] = x + h2 * jax.nn.sigmoid(h2)


def _node(part, cnt, nf, w3a, w3b, b3r, w4, b4r):
    grid = (_N // _BN,)
    return pl.pallas_call(
        _node_body,
        grid=grid,
        in_specs=[
            pl.BlockSpec((_NC, _BN, _H), lambda i: (0, i, 0)),
            pl.BlockSpec((_NC, _BN, 16), lambda i: (0, i, 0)),
            pl.BlockSpec((_BN, _H), lambda i: (i, 0)),
            pl.BlockSpec((_H, _H), lambda i: (0, 0)),
            pl.BlockSpec((_H, _H), lambda i: (0, 0)),
            pl.BlockSpec((1, _H), lambda i: (0, 0)),
            pl.BlockSpec((_H, _H), lambda i: (0, 0)),
            pl.BlockSpec((1, _H), lambda i: (0, 0)),
        ],
        out_specs=pl.BlockSpec((_BN, _H), lambda i: (i, 0)),
        out_shape=jax.ShapeDtypeStruct((_N, _H), jnp.float32),
    )(part, cnt, nf, w3a, w3b, b3r, w4, b4r)


def kernel(node_features, frac_coords, lattices, edge_index, edge2graph,
           W1, b1, W2, b2, W3, b3, W4, b4):
    src = edge_index[0]
    dst = edge_index[1]
    lat9 = lattices.reshape(_G, 9)
    fc16 = jnp.pad(frac_coords, ((0, 0), (0, 13)))
    sigma = [v for cc in range(_H // 2) for v in (cc, _H // 2 + cc)]
    w1a = W1[:_H][:, sigma]
    w1b = W1[_H:2 * _H][:, sigma]
    w1c = W1[2 * _H:2 * _H + 9][:, sigma]
    w1dp = jnp.pad(W1[2 * _H + 9:], ((0, 13), (0, 0)))
    b1r = b1[jnp.array(sigma)].reshape(1, _H)
    b2r = b2.reshape(1, _H)
    b3r = b3.reshape(1, _H)
    b4r = b4.reshape(1, _H)

    p, q, rb = _pre(node_features, lat9, w1a, w1b, w1c, b1r)

    def _as_i32(x):
        n, h = x.shape
        return lax.bitcast_convert_type(x.reshape(n, h // 2, 2), jnp.int32)

    fci = lax.bitcast_convert_type(fc16, jnp.int32)
    p32, q32, rb32 = _as_i32(p), _as_i32(q), _as_i32(rb)
    ones16 = jnp.ones((_W, 16), jnp.float32)
    zc = jnp.zeros((_N, 16), jnp.float32)
    cnt = _aux_kernel()(src, ones16, zc)
    w2bf = W2.astype(jnp.bfloat16)
    w2a, w2b = w2bf[:_H // 2], w2bf[_H // 2:]

    part = jnp.zeros((_NC, _N, _H), jnp.float32)
    off = 0
    for n_k in _CHUNKS:
        sl = slice(off, off + n_k)
        ab, cx = _gather_kernel(n_k)(p32, q32, rb32, fci,
                                     src[sl], dst[sl], edge2graph[sl])
        ef = _edge(ab, cx, w1dp, w2a, w2b, b2r)
        part = _scatter_kernel(n_k)(ef, src[sl], part)
        off += n_k
    return _node(part, cnt, node_features, W3[:_H], W3[_H:], b3r, W4, b4r)


# speedup vs baseline: 1.1702x; 1.1702x over previous
import functools

import jax
import jax.numpy as jnp
from jax import lax
from jax.experimental import pallas as pl
from jax.experimental.pallas import tpu as pltpu
from jax.experimental.pallas import tpu_sc as plsc

_N = 10000
_E = 320000
_G = 256
_H = 128

_NC = 2
_NS = 16
_NW = _NC * _NS
_EPW = _E // _NW
_W = 80

_BE = 2560
_CHUNKS = (107520, 107520, 104960)
_BN = 1000


@functools.lru_cache(maxsize=None)
def _mesh():
    return plsc.VectorSubcoreMesh(core_axis_name="c", subcore_axis_name="s")


def _pre_body(nf_ref, lat9_ref, w1a_ref, w1b_ref, w1c_ref, b1_ref,
              p_ref, q_ref, rb_ref):
    x = nf_ref[...]
    p_ref[...] = jnp.dot(x, w1a_ref[...],
                         preferred_element_type=jnp.float32).astype(jnp.bfloat16)
    q_ref[...] = jnp.dot(x, w1b_ref[...],
                         preferred_element_type=jnp.float32).astype(jnp.bfloat16)

    @pl.when(pl.program_id(0) == 0)
    def _():
        lat9 = lat9_ref[...]
        cols = [lat9[:, k:k + 1] for k in range(9)]
        acc = jnp.broadcast_to(b1_ref[...], (_G, _H))
        for i in range(3):
            for j in range(3):
                ip = (cols[3 * i + 0] * cols[3 * j + 0]
                      + cols[3 * i + 1] * cols[3 * j + 1]
                      + cols[3 * i + 2] * cols[3 * j + 2])
                acc = acc + ip * w1c_ref[3 * i + j:3 * i + j + 1, :]
        rb_ref[...] = acc.astype(jnp.bfloat16)


def _pre(nf, lat9, w1a, w1b, w1c, b1r):
    grid = (_N // _BN,)
    return pl.pallas_call(
        _pre_body,
        grid=grid,
        in_specs=[
            pl.BlockSpec((_BN, _H), lambda i: (i, 0)),
            pl.BlockSpec((_G, 9), lambda i: (0, 0)),
            pl.BlockSpec((_H, _H), lambda i: (0, 0)),
            pl.BlockSpec((_H, _H), lambda i: (0, 0)),
            pl.BlockSpec((9, _H), lambda i: (0, 0)),
            pl.BlockSpec((1, _H), lambda i: (0, 0)),
        ],
        out_specs=[
            pl.BlockSpec((_BN, _H), lambda i: (i, 0)),
            pl.BlockSpec((_BN, _H), lambda i: (i, 0)),
            pl.BlockSpec((_G, _H), lambda i: (0, 0)),
        ],
        out_shape=[
            jax.ShapeDtypeStruct((_N, _H), jnp.bfloat16),
            jax.ShapeDtypeStruct((_N, _H), jnp.bfloat16),
            jax.ShapeDtypeStruct((_G, _H), jnp.bfloat16),
        ],
    )(nf, lat9, w1a, w1b, w1c, b1r)


def _gather_body(epw, nwin,
                 p_hbm, q_hbm, rb_hbm, fc_hbm, src_hbm, dst_hbm, e2g_hbm,
                 ab_hbm, cx_hbm,
                 si_v, di_v, gi_v,
                 a0_v, b0_v, c0_v, xi0_v, xj0_v,
                 a1_v, b1_v, c1_v, xi1_v, xj1_v,
                 sem0, sem1):
    wid = lax.axis_index("s") * _NC + lax.axis_index("c")
    base = wid * epw
    ab0_v = (a0_v, b0_v, c0_v, xi0_v, xj0_v)
    ab1_v = (a1_v, b1_v, c1_v, xi1_v, xj1_v)

    pltpu.sync_copy(src_hbm.at[pl.ds(base, epw)], si_v)
    pltpu.sync_copy(dst_hbm.at[pl.ds(base, epw)], di_v)
    pltpu.sync_copy(e2g_hbm.at[pl.ds(base, epw)], gi_v)

    def issue(w, bufs, sem):
        a_v, b_v, c_v, xi_v, xj_v = bufs
        sl = pl.ds(w * _W, _W)
        pltpu.async_copy(p_hbm.at[si_v.at[sl]], a_v, sem)
        pltpu.async_copy(q_hbm.at[di_v.at[sl]], b_v, sem)
        pltpu.async_copy(rb_hbm.at[gi_v.at[sl]], c_v, sem)
        pltpu.async_copy(fc_hbm.at[si_v.at[sl]], xi_v, sem)
        pltpu.async_copy(fc_hbm.at[di_v.at[sl]], xj_v, sem)

    def drain_store(w, bufs, sem):
        a_v, b_v, c_v, xi_v, xj_v = bufs
        sl = pl.ds(w * _W, _W)
        pltpu.make_async_copy(p_hbm.at[si_v.at[sl]], a_v, sem).wait()
        pltpu.make_async_copy(q_hbm.at[di_v.at[sl]], b_v, sem).wait()
        pltpu.make_async_copy(rb_hbm.at[gi_v.at[sl]], c_v, sem).wait()
        pltpu.make_async_copy(fc_hbm.at[si_v.at[sl]], xi_v, sem).wait()
        pltpu.make_async_copy(fc_hbm.at[di_v.at[sl]], xj_v, sem).wait()
        rows = pl.ds(base + w * _W, _W)
        pltpu.sync_copy(a_v, ab_hbm.at[rows, pl.ds(0, 64)])
        pltpu.sync_copy(b_v, ab_hbm.at[rows, pl.ds(64, 64)])
        pltpu.sync_copy(c_v, cx_hbm.at[rows, pl.ds(0, 64)])
        pltpu.sync_copy(xi_v, cx_hbm.at[rows, pl.ds(64, 16)])
        pltpu.sync_copy(xj_v, cx_hbm.at[rows, pl.ds(80, 16)])

    issue(0, ab0_v, sem0)

    if nwin % 2 == 1:
        @pl.loop(0, nwin - 1, step=2)
        def _(w):
            issue(w + 1, ab1_v, sem1)
            drain_store(w, ab0_v, sem0)
            issue(w + 2, ab0_v, sem0)
            drain_store(w + 1, ab1_v, sem1)

        drain_store(nwin - 1, ab0_v, sem0)
    else:
        @pl.loop(0, nwin - 2, step=2)
        def _(w):
            issue(w + 1, ab1_v, sem1)
            drain_store(w, ab0_v, sem0)
            issue(w + 2, ab0_v, sem0)
            drain_store(w + 1, ab1_v, sem1)

        issue(nwin - 1, ab1_v, sem1)
        drain_store(nwin - 2, ab0_v, sem0)
        drain_store(nwin - 1, ab1_v, sem1)


@functools.lru_cache(maxsize=None)
def _gather_kernel(n_edges):
    epw = n_edges // _NW
    nwin = epw // _W
    return pl.kernel(
        functools.partial(_gather_body, epw, nwin),
        mesh=_mesh(),
        out_type=(
            jax.ShapeDtypeStruct((n_edges, _H), jnp.int32),
            jax.ShapeDtypeStruct((n_edges, _H), jnp.int32),
        ),
        scratch_types=[
            pltpu.VMEM((epw,), jnp.int32),
            pltpu.VMEM((epw,), jnp.int32),
            pltpu.VMEM((epw,), jnp.int32),
            pltpu.VMEM((_W, 64), jnp.int32),
            pltpu.VMEM((_W, 64), jnp.int32),
            pltpu.VMEM((_W, 64), jnp.int32),
            pltpu.VMEM((_W, 16), jnp.int32),
            pltpu.VMEM((_W, 16), jnp.int32),
            pltpu.VMEM((_W, 64), jnp.int32),
            pltpu.VMEM((_W, 64), jnp.int32),
            pltpu.VMEM((_W, 64), jnp.int32),
            pltpu.VMEM((_W, 16), jnp.int32),
            pltpu.VMEM((_W, 16), jnp.int32),
            pltpu.SemaphoreType.DMA,
            pltpu.SemaphoreType.DMA,
        ],
        compiler_params=pltpu.CompilerParams(use_tc_tiling_on_sc=False),
    )


def _aux_body(src_hbm, ones_hbm, zc_hbm, cnt_hbm,
              si_v, ones_v, cacc_sh, sem):
    cid = lax.axis_index("c")
    sid = lax.axis_index("s")
    wid = sid * _NC + cid
    base = wid * _EPW

    @pl.when(sid == 0)
    def _():
        pltpu.sync_copy(zc_hbm, cacc_sh)
    pltpu.sync_copy(ones_hbm, ones_v)
    plsc.subcore_barrier()

    @pl.loop(0, _EPW, step=_W)
    def _(off):
        st = base + off
        pltpu.sync_copy(src_hbm.at[pl.ds(st, _W)], si_v)
        pltpu.sync_copy(ones_v, cacc_sh.at[si_v], add=True)

    plsc.subcore_barrier()

    @pl.when(sid == 0)
    def _():
        pltpu.sync_copy(cacc_sh, cnt_hbm.at[cid])


@functools.lru_cache(maxsize=None)
def _aux_kernel():
    return pl.kernel(
        _aux_body,
        mesh=_mesh(),
        out_type=jax.ShapeDtypeStruct((_NC, _N, 16), jnp.float32),
        scratch_types=[
            pltpu.VMEM((_W,), jnp.int32),
            pltpu.VMEM((_W, 16), jnp.float32),
            pltpu.VMEM_SHARED((_N, 16), jnp.float32),
            pltpu.SemaphoreType.DMA,
        ],
        compiler_params=pltpu.CompilerParams(use_tc_tiling_on_sc=False),
    )


def _edge_body(ab_ref, cx_ref, w1dp_ref, w2a_ref, w2b_ref, b2_ref, out_ref):
    ab = ab_ref[...]
    cx = cx_ref[...]
    mask = jnp.int32(-65536)
    lo_ab = lax.bitcast_convert_type(ab << 16, jnp.float32)
    hi_ab = lax.bitcast_convert_type(ab & mask, jnp.float32)
    lo_c = lax.bitcast_convert_type(cx << 16, jnp.float32)
    hi_c = lax.bitcast_convert_type(cx & mask, jnp.float32)
    xi = lax.bitcast_convert_type(cx[:, 64:80], jnp.float32)
    xj = lax.bitcast_convert_type(cx[:, 80:96], jnp.float32)
    d = xj - xi
    fd = jnp.where(d < 0.0, d + 1.0, d)
    fterm = jnp.dot(fd, w1dp_ref[...], preferred_element_type=jnp.float32)
    zl = (lo_ab[:, :64] + lo_ab[:, 64:] + lo_c[:, :64] + fterm[:, :64])
    zh = (hi_ab[:, :64] + hi_ab[:, 64:] + hi_c[:, :64] + fterm[:, 64:])
    ul = zl * jax.nn.sigmoid(zl)
    uh = zh * jax.nn.sigmoid(zh)
    h2 = (jnp.dot(ul.astype(jnp.bfloat16), w2a_ref[...],
                  preferred_element_type=jnp.float32)
          + jnp.dot(uh.astype(jnp.bfloat16), w2b_ref[...],
                    preferred_element_type=jnp.float32)
          + b2_ref[...])
    out_ref[...] = h2 * jax.nn.sigmoid(h2)


def _edge(ab, cx, w1dp, w2a, w2b, b2r):
    n_edges = ab.shape[0]
    grid = (n_edges // _BE,)
    return pl.pallas_call(
        _edge_body,
        grid=grid,
        in_specs=[
            pl.BlockSpec((_BE, _H), lambda i: (i, 0)),
            pl.BlockSpec((_BE, _H), lambda i: (i, 0)),
            pl.BlockSpec((16, _H), lambda i: (0, 0)),
            pl.BlockSpec((_H // 2, _H), lambda i: (0, 0)),
            pl.BlockSpec((_H // 2, _H), lambda i: (0, 0)),
            pl.BlockSpec((1, _H), lambda i: (0, 0)),
        ],
        out_specs=pl.BlockSpec((_BE, _H), lambda i: (i, 0)),
        out_shape=jax.ShapeDtypeStruct((n_edges, _H), jnp.float32),
    )(ab, cx, w1dp, w2a, w2b, b2r)


def _scatter_body(epw, ef_hbm, src_hbm, z_hbm, part_hbm, idx_v, rows_v,
                  acc_sh, sem):
    cid = lax.axis_index("c")
    sid = lax.axis_index("s")
    wid = sid * _NC + cid
    base = wid * epw

    @pl.when(sid == 0)
    def _():
        pltpu.sync_copy(z_hbm.at[cid], acc_sh)
    plsc.subcore_barrier()

    @pl.loop(0, epw, step=_W)
    def _(off):
        st = base + off
        pltpu.sync_copy(src_hbm.at[pl.ds(st, _W)], idx_v)
        pltpu.sync_copy(ef_hbm.at[pl.ds(st, _W)], rows_v)
        pltpu.sync_copy(rows_v, acc_sh.at[idx_v], add=True)

    plsc.subcore_barrier()

    @pl.when(sid == 0)
    def _():
        pltpu.sync_copy(acc_sh, part_hbm.at[cid])


@functools.lru_cache(maxsize=None)
def _scatter_kernel(n_edges):
    return pl.kernel(
        functools.partial(_scatter_body, n_edges // _NW),
        mesh=_mesh(),
        out_type=jax.ShapeDtypeStruct((_NC, _N, _H), jnp.float32),
        scratch_types=[
            pltpu.VMEM((_W,), jnp.int32),
            pltpu.VMEM((_W, _H), jnp.float32),
            pltpu.VMEM_SHARED((_N, _H), jnp.float32),
            pltpu.SemaphoreType.DMA,
        ],
    )


def _node_body(part_ref, cnt_ref, nf_ref, w3a_ref, w3b_ref, b3_ref, w4_ref,
               b4_ref, out_ref):
    s = part_ref[0] + part_ref[1]
    cnt = cnt_ref[0][:, 0:1] + cnt_ref[1][:, 0:1]
    agg = s / jnp.maximum(cnt, 1.0)
    x = nf_ref[...]
    h = (jnp.dot(x, w3a_ref[...], preferred_element_type=jnp.float32)
         + jnp.dot(agg, w3b_ref[...], preferred_element_type=jnp.float32)
         + b3_ref[...])
    u = h * jax.nn.sigmoid(h)
    h2 = jnp.dot(u, w4_ref[...], preferred_element_type=jnp.float32) + b4_ref[...]
    out_ref[...] = x + h2 * jax.nn.sigmoid(h2)


def _node(part, cnt, nf, w3a, w3b, b3r, w4, b4r):
    grid = (_N // _BN,)
    return pl.pallas_call(
        _node_body,
        grid=grid,
        in_specs=[
            pl.BlockSpec((_NC, _BN, _H), lambda i: (0, i, 0)),
            pl.BlockSpec((_NC, _BN, 16), lambda i: (0, i, 0)),
            pl.BlockSpec((_BN, _H), lambda i: (i, 0)),
            pl.BlockSpec((_H, _H), lambda i: (0, 0)),
            pl.BlockSpec((_H, _H), lambda i: (0, 0)),
            pl.BlockSpec((1, _H), lambda i: (0, 0)),
            pl.BlockSpec((_H, _H), lambda i: (0, 0)),
            pl.BlockSpec((1, _H), lambda i: (0, 0)),
        ],
        out_specs=pl.BlockSpec((_BN, _H), lambda i: (i, 0)),
        out_shape=jax.ShapeDtypeStruct((_N, _H), jnp.float32),
    )(part, cnt, nf, w3a, w3b, b3r, w4, b4r)


def kernel(node_features, frac_coords, lattices, edge_index, edge2graph,
           W1, b1, W2, b2, W3, b3, W4, b4):
    src = edge_index[0]
    dst = edge_index[1]
    lat9 = lattices.reshape(_G, 9)
    fc16 = jnp.pad(frac_coords, ((0, 0), (0, 13)))
    sigma = [v for cc in range(_H // 2) for v in (cc, _H // 2 + cc)]
    w1a = W1[:_H][:, sigma]
    w1b = W1[_H:2 * _H][:, sigma]
    w1c = W1[2 * _H:2 * _H + 9][:, sigma]
    w1dp = jnp.pad(W1[2 * _H + 9:], ((0, 13), (0, 0)))
    b1r = b1[jnp.array(sigma)].reshape(1, _H)
    b2r = b2.reshape(1, _H)
    b3r = b3.reshape(1, _H)
    b4r = b4.reshape(1, _H)

    p, q, rb = _pre(node_features, lat9, w1a, w1b, w1c, b1r)

    def _as_i32(x):
        n, h = x.shape
        return lax.bitcast_convert_type(x.reshape(n, h // 2, 2), jnp.int32)

    fci = lax.bitcast_convert_type(fc16, jnp.int32)
    p32, q32, rb32 = _as_i32(p), _as_i32(q), _as_i32(rb)
    ones16 = jnp.ones((_W, 16), jnp.float32)
    zc = jnp.zeros((_N, 16), jnp.float32)
    cnt = _aux_kernel()(src, ones16, zc)
    w2bf = W2.astype(jnp.bfloat16)
    w2a, w2b = w2bf[:_H // 2], w2bf[_H // 2:]

    part = jnp.zeros((_NC, _N, _H), jnp.float32)
    off = 0
    for n_k in _CHUNKS:
        sl = slice(off, off + n_k)
        ab, cx = _gather_kernel(n_k)(p32, q32, rb32, fci,
                                     src[sl], dst[sl], edge2graph[sl])
        ef = _edge(ab, cx, w1dp, w2a, w2b, b2r)
        part = _scatter_kernel(n_k)(ef, src[sl], part)
        off += n_k
    return _node(part, cnt, node_features, W3[:_H], W3[_H:], b3r, W4, b4r)


# speedup vs baseline: 1.1883x; 1.0155x over previous
import functools

import jax
import jax.numpy as jnp
from jax import lax
from jax.experimental import pallas as pl
from jax.experimental.pallas import tpu as pltpu
from jax.experimental.pallas import tpu_sc as plsc

_N = 10000
_E = 320000
_G = 256
_H = 128

_NC = 2
_NS = 16
_NW = _NC * _NS
_EPW = _E // _NW
_W = 80

_BE = 2560
_CHUNKS = (163840, 156160)
_BN = 1000


@functools.lru_cache(maxsize=None)
def _mesh():
    return plsc.VectorSubcoreMesh(core_axis_name="c", subcore_axis_name="s")


def _pre_body(nf_ref, lat9_ref, w1a_ref, w1b_ref, w1c_ref, b1_ref,
              p_ref, q_ref, rb_ref):
    x = nf_ref[...]
    p_ref[...] = jnp.dot(x, w1a_ref[...],
                         preferred_element_type=jnp.float32).astype(jnp.bfloat16)
    q_ref[...] = jnp.dot(x, w1b_ref[...],
                         preferred_element_type=jnp.float32).astype(jnp.bfloat16)

    @pl.when(pl.program_id(0) == 0)
    def _():
        lat9 = lat9_ref[...]
        cols = [lat9[:, k:k + 1] for k in range(9)]
        acc = jnp.broadcast_to(b1_ref[...], (_G, _H))
        for i in range(3):
            for j in range(3):
                ip = (cols[3 * i + 0] * cols[3 * j + 0]
                      + cols[3 * i + 1] * cols[3 * j + 1]
                      + cols[3 * i + 2] * cols[3 * j + 2])
                acc = acc + ip * w1c_ref[3 * i + j:3 * i + j + 1, :]
        rb_ref[...] = acc.astype(jnp.bfloat16)


def _pre(nf, lat9, w1a, w1b, w1c, b1r):
    grid = (_N // _BN,)
    return pl.pallas_call(
        _pre_body,
        grid=grid,
        in_specs=[
            pl.BlockSpec((_BN, _H), lambda i: (i, 0)),
            pl.BlockSpec((_G, 9), lambda i: (0, 0)),
            pl.BlockSpec((_H, _H), lambda i: (0, 0)),
            pl.BlockSpec((_H, _H), lambda i: (0, 0)),
            pl.BlockSpec((9, _H), lambda i: (0, 0)),
            pl.BlockSpec((1, _H), lambda i: (0, 0)),
        ],
        out_specs=[
            pl.BlockSpec((_BN, _H), lambda i: (i, 0)),
            pl.BlockSpec((_BN, _H), lambda i: (i, 0)),
            pl.BlockSpec((_G, _H), lambda i: (0, 0)),
        ],
        out_shape=[
            jax.ShapeDtypeStruct((_N, _H), jnp.bfloat16),
            jax.ShapeDtypeStruct((_N, _H), jnp.bfloat16),
            jax.ShapeDtypeStruct((_G, _H), jnp.bfloat16),
        ],
    )(nf, lat9, w1a, w1b, w1c, b1r)


def _gather_body(epw, nwin,
                 p_hbm, q_hbm, rb_hbm, fc_hbm, src_hbm, dst_hbm, e2g_hbm,
                 ab_hbm, cx_hbm,
                 si_v, di_v, gi_v,
                 a0_v, b0_v, c0_v, xi0_v, xj0_v,
                 a1_v, b1_v, c1_v, xi1_v, xj1_v,
                 sem0, sem1):
    wid = lax.axis_index("s") * _NC + lax.axis_index("c")
    base = wid * epw
    ab0_v = (a0_v, b0_v, c0_v, xi0_v, xj0_v)
    ab1_v = (a1_v, b1_v, c1_v, xi1_v, xj1_v)

    pltpu.sync_copy(src_hbm.at[pl.ds(base, epw)], si_v)
    pltpu.sync_copy(dst_hbm.at[pl.ds(base, epw)], di_v)
    pltpu.sync_copy(e2g_hbm.at[pl.ds(base, epw)], gi_v)

    def issue(w, bufs, sem):
        a_v, b_v, c_v, xi_v, xj_v = bufs
        sl = pl.ds(w * _W, _W)
        pltpu.async_copy(p_hbm.at[si_v.at[sl]], a_v, sem)
        pltpu.async_copy(q_hbm.at[di_v.at[sl]], b_v, sem)
        pltpu.async_copy(rb_hbm.at[gi_v.at[sl]], c_v, sem)
        pltpu.async_copy(fc_hbm.at[si_v.at[sl]], xi_v, sem)
        pltpu.async_copy(fc_hbm.at[di_v.at[sl]], xj_v, sem)

    def drain_store(w, bufs, sem):
        a_v, b_v, c_v, xi_v, xj_v = bufs
        sl = pl.ds(w * _W, _W)
        pltpu.make_async_copy(p_hbm.at[si_v.at[sl]], a_v, sem).wait()
        pltpu.make_async_copy(q_hbm.at[di_v.at[sl]], b_v, sem).wait()
        pltpu.make_async_copy(rb_hbm.at[gi_v.at[sl]], c_v, sem).wait()
        pltpu.make_async_copy(fc_hbm.at[si_v.at[sl]], xi_v, sem).wait()
        pltpu.make_async_copy(fc_hbm.at[di_v.at[sl]], xj_v, sem).wait()
        rows = pl.ds(base + w * _W, _W)
        pltpu.sync_copy(a_v, ab_hbm.at[rows, pl.ds(0, 64)])
        pltpu.sync_copy(b_v, ab_hbm.at[rows, pl.ds(64, 64)])
        pltpu.sync_copy(c_v, cx_hbm.at[rows, pl.ds(0, 64)])
        pltpu.sync_copy(xi_v, cx_hbm.at[rows, pl.ds(64, 16)])
        pltpu.sync_copy(xj_v, cx_hbm.at[rows, pl.ds(80, 16)])

    issue(0, ab0_v, sem0)

    if nwin % 2 == 1:
        @pl.loop(0, nwin - 1, step=2)
        def _(w):
            issue(w + 1, ab1_v, sem1)
            drain_store(w, ab0_v, sem0)
            issue(w + 2, ab0_v, sem0)
            drain_store(w + 1, ab1_v, sem1)

        drain_store(nwin - 1, ab0_v, sem0)
    else:
        @pl.loop(0, nwin - 2, step=2)
        def _(w):
            issue(w + 1, ab1_v, sem1)
            drain_store(w, ab0_v, sem0)
            issue(w + 2, ab0_v, sem0)
            drain_store(w + 1, ab1_v, sem1)

        issue(nwin - 1, ab1_v, sem1)
        drain_store(nwin - 2, ab0_v, sem0)
        drain_store(nwin - 1, ab1_v, sem1)


@functools.lru_cache(maxsize=None)
def _gather_kernel(n_edges):
    epw = n_edges // _NW
    nwin = epw // _W
    return pl.kernel(
        functools.partial(_gather_body, epw, nwin),
        mesh=_mesh(),
        out_type=(
            jax.ShapeDtypeStruct((n_edges, _H), jnp.int32),
            jax.ShapeDtypeStruct((n_edges, _H), jnp.int32),
        ),
        scratch_types=[
            pltpu.VMEM((epw,), jnp.int32),
            pltpu.VMEM((epw,), jnp.int32),
            pltpu.VMEM((epw,), jnp.int32),
            pltpu.VMEM((_W, 64), jnp.int32),
            pltpu.VMEM((_W, 64), jnp.int32),
            pltpu.VMEM((_W, 64), jnp.int32),
            pltpu.VMEM((_W, 16), jnp.int32),
            pltpu.VMEM((_W, 16), jnp.int32),
            pltpu.VMEM((_W, 64), jnp.int32),
            pltpu.VMEM((_W, 64), jnp.int32),
            pltpu.VMEM((_W, 64), jnp.int32),
            pltpu.VMEM((_W, 16), jnp.int32),
            pltpu.VMEM((_W, 16), jnp.int32),
            pltpu.SemaphoreType.DMA,
            pltpu.SemaphoreType.DMA,
        ],
        compiler_params=pltpu.CompilerParams(use_tc_tiling_on_sc=False),
    )


def _aux_body(src_hbm, ones_hbm, zc_hbm, cnt_hbm,
              si_v, ones_v, cacc_sh, sem):
    cid = lax.axis_index("c")
    sid = lax.axis_index("s")
    wid = sid * _NC + cid
    base = wid * _EPW

    @pl.when(sid == 0)
    def _():
        pltpu.sync_copy(zc_hbm, cacc_sh)
    pltpu.sync_copy(ones_hbm, ones_v)
    plsc.subcore_barrier()

    @pl.loop(0, _EPW, step=_W)
    def _(off):
        st = base + off
        pltpu.sync_copy(src_hbm.at[pl.ds(st, _W)], si_v)
        pltpu.sync_copy(ones_v, cacc_sh.at[si_v], add=True)

    plsc.subcore_barrier()

    @pl.when(sid == 0)
    def _():
        pltpu.sync_copy(cacc_sh, cnt_hbm.at[cid])


@functools.lru_cache(maxsize=None)
def _aux_kernel():
    return pl.kernel(
        _aux_body,
        mesh=_mesh(),
        out_type=jax.ShapeDtypeStruct((_NC, _N, 16), jnp.float32),
        scratch_types=[
            pltpu.VMEM((_W,), jnp.int32),
            pltpu.VMEM((_W, 16), jnp.float32),
            pltpu.VMEM_SHARED((_N, 16), jnp.float32),
            pltpu.SemaphoreType.DMA,
        ],
        compiler_params=pltpu.CompilerParams(use_tc_tiling_on_sc=False),
    )


def _edge_body(ab_ref, cx_ref, w1dp_ref, w2a_ref, w2b_ref, b2_ref, out_ref):
    ab = ab_ref[...]
    cx = cx_ref[...]
    mask = jnp.int32(-65536)
    lo_ab = lax.bitcast_convert_type(ab << 16, jnp.float32)
    hi_ab = lax.bitcast_convert_type(ab & mask, jnp.float32)
    lo_c = lax.bitcast_convert_type(cx << 16, jnp.float32)
    hi_c = lax.bitcast_convert_type(cx & mask, jnp.float32)
    xi = lax.bitcast_convert_type(cx[:, 64:80], jnp.float32)
    xj = lax.bitcast_convert_type(cx[:, 80:96], jnp.float32)
    d = xj - xi
    fd = jnp.where(d < 0.0, d + 1.0, d)
    fterm = jnp.dot(fd, w1dp_ref[...], preferred_element_type=jnp.float32)
    zl = (lo_ab[:, :64] + lo_ab[:, 64:] + lo_c[:, :64] + fterm[:, :64])
    zh = (hi_ab[:, :64] + hi_ab[:, 64:] + hi_c[:, :64] + fterm[:, 64:])
    ul = zl * jax.nn.sigmoid(zl)
    uh = zh * jax.nn.sigmoid(zh)
    h2 = (jnp.dot(ul.astype(jnp.bfloat16), w2a_ref[...],
                  preferred_element_type=jnp.float32)
          + jnp.dot(uh.astype(jnp.bfloat16), w2b_ref[...],
                    preferred_element_type=jnp.float32)
          + b2_ref[...])
    out_ref[...] = h2 * jax.nn.sigmoid(h2)


def _edge(ab, cx, w1dp, w2a, w2b, b2r):
    n_edges = ab.shape[0]
    grid = (n_edges // _BE,)
    return pl.pallas_call(
        _edge_body,
        grid=grid,
        in_specs=[
            pl.BlockSpec((_BE, _H), lambda i: (i, 0)),
            pl.BlockSpec((_BE, _H), lambda i: (i, 0)),
            pl.BlockSpec((16, _H), lambda i: (0, 0)),
            pl.BlockSpec((_H // 2, _H), lambda i: (0, 0)),
            pl.BlockSpec((_H // 2, _H), lambda i: (0, 0)),
            pl.BlockSpec((1, _H), lambda i: (0, 0)),
        ],
        out_specs=pl.BlockSpec((_BE, _H), lambda i: (i, 0)),
        out_shape=jax.ShapeDtypeStruct((n_edges, _H), jnp.float32),
    )(ab, cx, w1dp, w2a, w2b, b2r)


def _scatter_body(epw, ef_hbm, src_hbm, z_hbm, part_hbm, idx_v, rows_v,
                  acc_sh, sem):
    cid = lax.axis_index("c")
    sid = lax.axis_index("s")
    wid = sid * _NC + cid
    base = wid * epw

    @pl.when(sid == 0)
    def _():
        pltpu.sync_copy(z_hbm.at[cid], acc_sh)
    plsc.subcore_barrier()

    @pl.loop(0, epw, step=_W)
    def _(off):
        st = base + off
        pltpu.sync_copy(src_hbm.at[pl.ds(st, _W)], idx_v)
        pltpu.sync_copy(ef_hbm.at[pl.ds(st, _W)], rows_v)
        pltpu.sync_copy(rows_v, acc_sh.at[idx_v], add=True)

    plsc.subcore_barrier()

    @pl.when(sid == 0)
    def _():
        pltpu.sync_copy(acc_sh, part_hbm.at[cid])


@functools.lru_cache(maxsize=None)
def _scatter_kernel(n_edges):
    return pl.kernel(
        functools.partial(_scatter_body, n_edges // _NW),
        mesh=_mesh(),
        out_type=jax.ShapeDtypeStruct((_NC, _N, _H), jnp.float32),
        scratch_types=[
            pltpu.VMEM((_W,), jnp.int32),
            pltpu.VMEM((_W, _H), jnp.float32),
            pltpu.VMEM_SHARED((_N, _H), jnp.float32),
            pltpu.SemaphoreType.DMA,
        ],
    )


def _node_body(part_ref, cnt_ref, nf_ref, w3a_ref, w3b_ref, b3_ref, w4_ref,
               b4_ref, out_ref):
    s = part_ref[0] + part_ref[1]
    cnt = cnt_ref[0][:, 0:1] + cnt_ref[1][:, 0:1]
    agg = s / jnp.maximum(cnt, 1.0)
    x = nf_ref[...]
    h = (jnp.dot(x, w3a_ref[...], preferred_element_type=jnp.float32)
         + jnp.dot(agg, w3b_ref[...], preferred_element_type=jnp.float32)
         + b3_ref[...])
    u = h * jax.nn.sigmoid(h)
    h2 = jnp.dot(u, w4_ref[...], preferred_element_type=jnp.float32) + b4_ref[...]
    out_ref[...] = x + h2 * jax.nn.sigmoid(h2)


def _node(part, cnt, nf, w3a, w3b, b3r, w4, b4r):
    grid = (_N // _BN,)
    return pl.pallas_call(
        _node_body,
        grid=grid,
        in_specs=[
            pl.BlockSpec((_NC, _BN, _H), lambda i: (0, i, 0)),
            pl.BlockSpec((_NC, _BN, 16), lambda i: (0, i, 0)),
            pl.BlockSpec((_BN, _H), lambda i: (i, 0)),
            pl.BlockSpec((_H, _H), lambda i: (0, 0)),
            pl.BlockSpec((_H, _H), lambda i: (0, 0)),
            pl.BlockSpec((1, _H), lambda i: (0, 0)),
            pl.BlockSpec((_H, _H), lambda i: (0, 0)),
            pl.BlockSpec((1, _H), lambda i: (0, 0)),
        ],
        out_specs=pl.BlockSpec((_BN, _H), lambda i: (i, 0)),
        out_shape=jax.ShapeDtypeStruct((_N, _H), jnp.float32),
    )(part, cnt, nf, w3a, w3b, b3r, w4, b4r)


def kernel(node_features, frac_coords, lattices, edge_index, edge2graph,
           W1, b1, W2, b2, W3, b3, W4, b4):
    src = edge_index[0]
    dst = edge_index[1]
    lat9 = lattices.reshape(_G, 9)
    fc16 = jnp.pad(frac_coords, ((0, 0), (0, 13)))
    sigma = [v for cc in range(_H // 2) for v in (cc, _H // 2 + cc)]
    w1a = W1[:_H][:, sigma]
    w1b = W1[_H:2 * _H][:, sigma]
    w1c = W1[2 * _H:2 * _H + 9][:, sigma]
    w1dp = jnp.pad(W1[2 * _H + 9:], ((0, 13), (0, 0)))
    b1r = b1[jnp.array(sigma)].reshape(1, _H)
    b2r = b2.reshape(1, _H)
    b3r = b3.reshape(1, _H)
    b4r = b4.reshape(1, _H)

    p, q, rb = _pre(node_features, lat9, w1a, w1b, w1c, b1r)

    def _as_i32(x):
        n, h = x.shape
        return lax.bitcast_convert_type(x.reshape(n, h // 2, 2), jnp.int32)

    fci = lax.bitcast_convert_type(fc16, jnp.int32)
    p32, q32, rb32 = _as_i32(p), _as_i32(q), _as_i32(rb)
    ones16 = jnp.ones((_W, 16), jnp.float32)
    zc = jnp.zeros((_N, 16), jnp.float32)
    cnt = _aux_kernel()(src, ones16, zc)
    w2bf = W2.astype(jnp.bfloat16)
    w2a, w2b = w2bf[:_H // 2], w2bf[_H // 2:]

    part = jnp.zeros((_NC, _N, _H), jnp.float32)
    off = 0
    for n_k in _CHUNKS:
        sl = slice(off, off + n_k)
        ab, cx = _gather_kernel(n_k)(p32, q32, rb32, fci,
                                     src[sl], dst[sl], edge2graph[sl])
        ef = _edge(ab, cx, w1dp, w2a, w2b, b2r)
        part = _scatter_kernel(n_k)(ef, src[sl], part)
        off += n_k
    return _node(part, cnt, node_features, W3[:_H], W3[_H:], b3r, W4, b4r)
